# baseline (device time: 26486 ns/iter reference)
import jax
import jax.numpy as jnp
from jax import lax
from jax.experimental import pallas as pl
from jax.experimental.pallas import tpu as pltpu

N_DEV = 16
N_TOK = 512
D_IN = 256
D_OUT = 512
N_EXP = 64
E_LOCAL = 4
ROWS_PER = N_TOK // N_DEV


def kernel(x, router_W, route_idx, expert_W):
    def body(x_ref, rw_ref, idx_ref, ew_ref, out_ref,
             partial_ref, recv_ref, send_sems, recv_sems):
        my_pos = lax.axis_index("i")

        xf = x_ref[:, :]
        scores = jnp.dot(
            xf.astype(jnp.bfloat16),
            rw_ref[:, :].astype(jnp.bfloat16),
            preferred_element_type=jnp.float32,
        )
        s_max = jnp.max(scores, axis=-1, keepdims=True)
        exps = jnp.exp(scores - s_max)
        lane = lax.broadcasted_iota(jnp.int32, (N_TOK, N_EXP), 1)
        sel = jnp.logical_or(
            lane == idx_ref[:, 0:1], lane == idx_ref[:, 1:2]
        ).astype(jnp.float32)
        w_sel = sel * exps
        w = w_sel / jnp.sum(w_sel, axis=-1, keepdims=True)

        row64 = lax.broadcasted_iota(jnp.int32, (N_EXP, E_LOCAL), 0)
        col4 = lax.broadcasted_iota(jnp.int32, (N_EXP, E_LOCAL), 1)
        sel_local = (row64 == my_pos * E_LOCAL + col4).astype(jnp.float32)
        g_local = jnp.dot(w, sel_local, preferred_element_type=jnp.float32)

        x_bf = xf.astype(jnp.bfloat16)
        acc = jnp.zeros((N_TOK, D_OUT), jnp.float32)
        for le in range(E_LOCAL):
            y = jnp.dot(
                x_bf,
                ew_ref[le, :, :].astype(jnp.bfloat16),
                preferred_element_type=jnp.float32,
            )
            acc = acc + y * g_local[:, le:le + 1]
        partial_ref[:, :] = acc

        rdmas = []
        for d in range(1, N_DEV):
            tgt = (my_pos + d) % N_DEV
            rdma = pltpu.make_async_remote_copy(
                src_ref=partial_ref.at[pl.ds(tgt * ROWS_PER, ROWS_PER), :],
                dst_ref=recv_ref.at[d - 1],
                send_sem=send_sems.at[d - 1],
                recv_sem=recv_sems.at[d - 1],
                device_id=(tgt,),
                device_id_type=pl.DeviceIdType.MESH,
            )
            rdma.start()
            rdmas.append(rdma)

        total = partial_ref[pl.ds(my_pos * ROWS_PER, ROWS_PER), :]
        for d in range(1, N_DEV):
            rdmas[d - 1].wait_recv()
            total = total + recv_ref[d - 1, :, :]
        out_ref[:, :] = total
        for d in range(1, N_DEV):
            rdmas[d - 1].wait_send()

    return pl.pallas_call(
        body,
        out_shape=jax.ShapeDtypeStruct((ROWS_PER, D_OUT), jnp.float32),
        in_specs=[
            pl.BlockSpec(memory_space=pltpu.VMEM),
            pl.BlockSpec(memory_space=pltpu.VMEM),
            pl.BlockSpec(memory_space=pltpu.VMEM),
            pl.BlockSpec(memory_space=pltpu.VMEM),
        ],
        out_specs=pl.BlockSpec(memory_space=pltpu.VMEM),
        scratch_shapes=[
            pltpu.VMEM((N_TOK, D_OUT), jnp.float32),
            pltpu.VMEM((N_DEV - 1, ROWS_PER, D_OUT), jnp.float32),
            pltpu.SemaphoreType.DMA((N_DEV - 1,)),
            pltpu.SemaphoreType.DMA((N_DEV - 1,)),
        ],
    )(x, router_W, route_idx, expert_W)


# device time: 22166 ns/iter; 1.1949x vs baseline; 1.1949x over previous
import jax
import jax.numpy as jnp
from jax import lax
from jax.experimental import pallas as pl
from jax.experimental.pallas import tpu as pltpu

N_DEV = 16
N_TOK = 512
D_IN = 256
D_OUT = 512
N_EXP = 64
E_LOCAL = 4
ROWS_PER = N_TOK // N_DEV


def kernel(x, router_W, route_idx, expert_W):
    def body(x_ref, rw_ref, idx_ref, ew_ref, out_ref,
             partial_ref, recv_ref, send_sems, recv_sems):
        my_pos = lax.axis_index("i")

        xf = x_ref[:, :]
        scores = jnp.dot(
            xf.astype(jnp.bfloat16),
            rw_ref[:, :].astype(jnp.bfloat16),
            preferred_element_type=jnp.float32,
        )
        s_max = jnp.max(scores, axis=-1, keepdims=True)
        exps = jnp.exp(scores - s_max)
        lane = lax.broadcasted_iota(jnp.int32, (N_TOK, N_EXP), 1)
        sel = jnp.logical_or(
            lane == idx_ref[:, 0:1], lane == idx_ref[:, 1:2]
        ).astype(jnp.float32)
        w_sel = sel * exps
        w = w_sel / jnp.sum(w_sel, axis=-1, keepdims=True)

        row64 = lax.broadcasted_iota(jnp.int32, (N_EXP, E_LOCAL), 0)
        col4 = lax.broadcasted_iota(jnp.int32, (N_EXP, E_LOCAL), 1)
        sel_local = (row64 == my_pos * E_LOCAL + col4).astype(jnp.float32)
        g_local = jnp.dot(w, sel_local, preferred_element_type=jnp.float32)

        x_bf = xf.astype(jnp.bfloat16)
        acc = jnp.zeros((N_TOK, D_OUT), jnp.float32)
        for le in range(E_LOCAL):
            y = jnp.dot(
                x_bf,
                ew_ref[le, :, :].astype(jnp.bfloat16),
                preferred_element_type=jnp.float32,
            )
            acc = acc + y * g_local[:, le:le + 1]
        partial_ref[:, :] = acc.astype(jnp.bfloat16)

        rdmas = []
        for d in range(1, N_DEV):
            tgt = (my_pos + d) % N_DEV
            rdma = pltpu.make_async_remote_copy(
                src_ref=partial_ref.at[pl.ds(tgt * ROWS_PER, ROWS_PER), :],
                dst_ref=recv_ref.at[d - 1],
                send_sem=send_sems.at[d - 1],
                recv_sem=recv_sems.at[d - 1],
                device_id=(tgt,),
                device_id_type=pl.DeviceIdType.MESH,
            )
            rdma.start()
            rdmas.append(rdma)

        total = partial_ref[pl.ds(my_pos * ROWS_PER, ROWS_PER), :].astype(
            jnp.float32
        )
        for d in range(1, N_DEV):
            rdmas[d - 1].wait_recv()
            total = total + recv_ref[d - 1, :, :].astype(jnp.float32)
        out_ref[:, :] = total
        for d in range(1, N_DEV):
            rdmas[d - 1].wait_send()

    return pl.pallas_call(
        body,
        out_shape=jax.ShapeDtypeStruct((ROWS_PER, D_OUT), jnp.float32),
        in_specs=[
            pl.BlockSpec(memory_space=pltpu.VMEM),
            pl.BlockSpec(memory_space=pltpu.VMEM),
            pl.BlockSpec(memory_space=pltpu.VMEM),
            pl.BlockSpec(memory_space=pltpu.VMEM),
        ],
        out_specs=pl.BlockSpec(memory_space=pltpu.VMEM),
        scratch_shapes=[
            pltpu.VMEM((N_TOK, D_OUT), jnp.bfloat16),
            pltpu.VMEM((N_DEV - 1, ROWS_PER, D_OUT), jnp.bfloat16),
            pltpu.SemaphoreType.DMA((N_DEV - 1,)),
            pltpu.SemaphoreType.DMA((N_DEV - 1,)),
        ],
    )(x, router_W, route_idx, expert_W)
